# baseline (speedup 1.0000x reference)
import functools

import jax
import jax.numpy as jnp
from jax import lax
from jax.experimental import pallas as pl
from jax.experimental.pallas import tpu as pltpu
from jax.experimental.pallas import tpu_sc as plsc

VOCAB = 1000000
EI = 64
ED = 128
B = 16384
L = 200
HALF = L // 2
LP = 256
NC, NS = 2, 16
NW = NC * NS
BPW = B // NW
GROUP = 64
NGROUPS = BPW // GROUP
NBUF = 4


def _pooled_sc(x2, emb_table):
    mesh = plsc.VectorSubcoreMesh(core_axis_name="c", subcore_axis_name="s")

    @functools.partial(
        pl.kernel,
        out_type=jax.ShapeDtypeStruct((B, EI), jnp.float32),
        mesh=mesh,
        scratch_types=[
            pltpu.VMEM((GROUP, LP), jnp.int32),
            pltpu.VMEM((NBUF * L, EI), jnp.float32),
            pltpu.VMEM((BPW, EI), jnp.float32),
            pltpu.SemaphoreType.DMA((NBUF,)),
        ],
        compiler_params=pltpu.CompilerParams(use_tc_tiling_on_sc=False),
    )
    def _pool(x_hbm, table_hbm, out_hbm, idx_v, rows_v, out_v, sems):
        wid = lax.axis_index("s") * NC + lax.axis_index("c")
        base = wid * BPW

        def fire(r):
            buf = lax.rem(r, NBUF)
            return pltpu.async_copy(
                table_hbm.at[idx_v.at[r, pl.ds(0, L)]],
                rows_v.at[pl.ds(buf * L, L)], sems.at[buf])

        for g in range(NGROUPS):
            pltpu.sync_copy(
                x_hbm.at[pl.ds(base + g * GROUP, GROUP)], idx_v)
            for r0 in range(NBUF - 1):
                fire(r0)

            def row_body(r, _):
                buf = lax.rem(r, NBUF)
                pltpu.make_async_copy(
                    table_hbm.at[idx_v.at[r, pl.ds(0, L)]],
                    rows_v.at[pl.ds(buf * L, L)], sems.at[buf]).wait()

                @pl.when(r + (NBUF - 1) < GROUP)
                def _():
                    fire(r + (NBUF - 1))

                b0 = buf * L

                def acc_body(t, accs):
                    out = []
                    for c in range(4):
                        a = accs[c]
                        a = a + rows_v[b0 + t, pl.ds(c * 16, 16)]
                        a = a + rows_v[b0 + HALF + t, pl.ds(c * 16, 16)]
                        out.append(a)
                    return tuple(out)

                zeros = tuple(jnp.zeros((16,), jnp.float32) for _ in range(4))
                accs = lax.fori_loop(0, HALF, acc_body, zeros, unroll=2)
                lr = g * GROUP + r
                for c in range(4):
                    out_v[lr, pl.ds(c * 16, 16)] = accs[c]
                return 0

            lax.fori_loop(0, GROUP, row_body, 0)

        pltpu.sync_copy(out_v, out_hbm.at[pl.ds(base, BPW)])

    return _pool(x2, emb_table)


CW = 4096
NCONV = (VOCAB + CW - 1) // CW
VOCAB2 = NCONV * CW


def _conv_body(t_ref, o_ref):
    t = t_ref[...].T
    o_ref[...] = jnp.concatenate([t[: CW // 2], t[CW // 2:]], axis=1)


def _convert_table(tT):
    out = pl.pallas_call(
        _conv_body,
        grid=(NCONV,),
        in_specs=[pl.BlockSpec((EI, CW), lambda s: (0, s))],
        out_specs=pl.BlockSpec((CW // 2, 128), lambda s: (s, 0)),
        out_shape=jax.ShapeDtypeStruct((VOCAB2 // 2, 128), jnp.float32),
    )(tT)
    return out.reshape(VOCAB2, EI)


def _remap_idx(i):
    h = CW // 2
    return (i // CW) * CW + 2 * (i % h) + (i // h) % 2


def _fc_body(p_ref, w_ref, b_ref, o_ref):
    e = p_ref[...] * (1.0 / L)
    y = jnp.dot(e, w_ref[...], preferred_element_type=jnp.float32) + b_ref[...]
    n = jnp.sqrt(jnp.sum(y * y, axis=-1, keepdims=True))
    o_ref[...] = y / jnp.maximum(n, 1e-12)


def _fc_norm(pooled, fc_W, fc_b2):
    TB = 1024
    return pl.pallas_call(
        _fc_body,
        grid=(B // TB,),
        in_specs=[
            pl.BlockSpec((TB, EI), lambda i: (i, 0)),
            pl.BlockSpec((EI, ED), lambda i: (0, 0)),
            pl.BlockSpec((1, ED), lambda i: (0, 0)),
        ],
        out_specs=pl.BlockSpec((TB, ED), lambda i: (i, 0)),
        out_shape=jax.ShapeDtypeStruct((B, ED), jnp.float32),
    )(pooled, fc_W, fc_b2)


def kernel(x, emb_table, fc_W, fc_b):
    xp = jnp.pad(_remap_idx(x.astype(jnp.int32)), ((0, 0), (0, LP - L)))
    pooled = _pooled_sc(xp, _convert_table(emb_table.T))
    return _fc_norm(pooled, fc_W, fc_b.reshape(1, ED))


# speedup vs baseline: 1.1054x; 1.1054x over previous
import functools

import jax
import jax.numpy as jnp
from jax import lax
from jax.experimental import pallas as pl
from jax.experimental.pallas import tpu as pltpu
from jax.experimental.pallas import tpu_sc as plsc

VOCAB = 1000000
EI = 64
ED = 128
B = 16384
L = 200
HALF = L // 2
LP = 256
NC, NS = 2, 16
NW = NC * NS
BPW = B // NW
GROUP = 64
NGROUPS = BPW // GROUP
NBUF = 4


def _pooled_sc(x2, emb_table):
    mesh = plsc.VectorSubcoreMesh(core_axis_name="c", subcore_axis_name="s")

    @functools.partial(
        pl.kernel,
        out_type=jax.ShapeDtypeStruct((B, EI), jnp.float32),
        mesh=mesh,
        scratch_types=[
            pltpu.VMEM((GROUP, LP), jnp.int32),
            pltpu.VMEM((NBUF * L, EI), jnp.float32),
            pltpu.VMEM((BPW, EI), jnp.float32),
            pltpu.SemaphoreType.DMA((NBUF,)),
        ],
        compiler_params=pltpu.CompilerParams(use_tc_tiling_on_sc=False),
    )
    def _pool(x_hbm, table_hbm, out_hbm, idx_v, rows_v, out_v, sems):
        wid = lax.axis_index("s") * NC + lax.axis_index("c")
        base = wid * BPW

        def fire(r):
            buf = lax.rem(r, NBUF)
            return pltpu.async_copy(
                table_hbm.at[idx_v.at[r, pl.ds(0, L)]],
                rows_v.at[pl.ds(buf * L, L)], sems.at[buf])

        for g in range(NGROUPS):
            pltpu.sync_copy(
                x_hbm.at[pl.ds(base + g * GROUP, GROUP)], idx_v)
            for r0 in range(NBUF - 1):
                fire(r0)

            def row_body(r, _):
                buf = lax.rem(r, NBUF)
                pltpu.make_async_copy(
                    table_hbm.at[idx_v.at[r, pl.ds(0, L)]],
                    rows_v.at[pl.ds(buf * L, L)], sems.at[buf]).wait()

                @pl.when(r + (NBUF - 1) < GROUP)
                def _():
                    fire(r + (NBUF - 1))

                b0 = buf * L

                def acc_body(t, accs):
                    out = []
                    for c in range(4):
                        a = accs[c]
                        a = a + rows_v[b0 + t, pl.ds(c * 16, 16)]
                        a = a + rows_v[b0 + HALF + t, pl.ds(c * 16, 16)]
                        out.append(a)
                    return tuple(out)

                zeros = tuple(jnp.zeros((16,), jnp.float32) for _ in range(4))
                accs = lax.fori_loop(0, HALF, acc_body, zeros, unroll=2)
                lr = g * GROUP + r
                for c in range(4):
                    out_v[lr, pl.ds(c * 16, 16)] = accs[c]
                return 0

            lax.fori_loop(0, GROUP, row_body, 0)

        pltpu.sync_copy(out_v, out_hbm.at[pl.ds(base, BPW)])

    return _pool(x2, emb_table)


CW = 8192
NCONV = (VOCAB + CW - 1) // CW
VOCAB2 = NCONV * CW


def _conv_body(t_ref, o_ref):
    t = t_ref[...].T
    o_ref[...] = jnp.concatenate([t[: CW // 2], t[CW // 2:]], axis=1)


def _convert_table(tT):
    out = pl.pallas_call(
        _conv_body,
        grid=(NCONV,),
        in_specs=[pl.BlockSpec((EI, CW), lambda s: (0, s))],
        out_specs=pl.BlockSpec((CW // 2, 128), lambda s: (s, 0)),
        out_shape=jax.ShapeDtypeStruct((VOCAB2 // 2, 128), jnp.float32),
    )(tT)
    return out.reshape(VOCAB2, EI)


def _remap_idx(i):
    h = CW // 2
    return (i // CW) * CW + 2 * (i % h) + (i // h) % 2


def _fc_body(p_ref, w_ref, b_ref, o_ref):
    e = p_ref[...] * (1.0 / L)
    y = jnp.dot(e, w_ref[...], preferred_element_type=jnp.float32) + b_ref[...]
    n = jnp.sqrt(jnp.sum(y * y, axis=-1, keepdims=True))
    o_ref[...] = y / jnp.maximum(n, 1e-12)


def _fc_norm(pooled, fc_W, fc_b2):
    TB = 1024
    return pl.pallas_call(
        _fc_body,
        grid=(B // TB,),
        in_specs=[
            pl.BlockSpec((TB, EI), lambda i: (i, 0)),
            pl.BlockSpec((EI, ED), lambda i: (0, 0)),
            pl.BlockSpec((1, ED), lambda i: (0, 0)),
        ],
        out_specs=pl.BlockSpec((TB, ED), lambda i: (i, 0)),
        out_shape=jax.ShapeDtypeStruct((B, ED), jnp.float32),
    )(pooled, fc_W, fc_b2)


def kernel(x, emb_table, fc_W, fc_b):
    xp = jnp.pad(_remap_idx(x.astype(jnp.int32)), ((0, 0), (0, LP - L)))
    pooled = _pooled_sc(xp, _convert_table(emb_table.T))
    return _fc_norm(pooled, fc_W, fc_b.reshape(1, ED))


# speedup vs baseline: 1.1631x; 1.0522x over previous
import functools

import jax
import jax.numpy as jnp
from jax import lax
from jax.experimental import pallas as pl
from jax.experimental.pallas import tpu as pltpu
from jax.experimental.pallas import tpu_sc as plsc

VOCAB = 1000000
EI = 64
ED = 128
B = 16384
L = 200
HALF = L // 2
LP = 256
NC, NS = 2, 16
NW = NC * NS
BPW = B // NW
GROUP = 64
NGROUPS = BPW // GROUP
NBUF = 4


def _pooled_sc(x2, emb_table):
    mesh = plsc.VectorSubcoreMesh(core_axis_name="c", subcore_axis_name="s")

    @functools.partial(
        pl.kernel,
        out_type=jax.ShapeDtypeStruct((B, EI), jnp.float32),
        mesh=mesh,
        scratch_types=[
            pltpu.VMEM((GROUP, LP), jnp.int32),
            pltpu.VMEM((NBUF * L, EI), jnp.float32),
            pltpu.VMEM((BPW, EI), jnp.float32),
            pltpu.SemaphoreType.DMA((NBUF,)),
        ],
        compiler_params=pltpu.CompilerParams(use_tc_tiling_on_sc=False),
    )
    def _pool(x_hbm, table_hbm, out_hbm, idx_v, rows_v, out_v, sems):
        wid = lax.axis_index("s") * NC + lax.axis_index("c")
        base = wid * BPW

        def fire(r):
            buf = lax.rem(r, NBUF)
            return pltpu.async_copy(
                table_hbm.at[idx_v.at[r, pl.ds(0, L)]],
                rows_v.at[pl.ds(buf * L, L)], sems.at[buf])

        for g in range(NGROUPS):
            pltpu.sync_copy(
                x_hbm.at[pl.ds(base + g * GROUP, GROUP)], idx_v)
            for r0 in range(NBUF - 1):
                fire(r0)

            def row_body(r, _):
                buf = lax.rem(r, NBUF)
                pltpu.make_async_copy(
                    table_hbm.at[idx_v.at[r, pl.ds(0, L)]],
                    rows_v.at[pl.ds(buf * L, L)], sems.at[buf]).wait()

                @pl.when(r + (NBUF - 1) < GROUP)
                def _():
                    fire(r + (NBUF - 1))

                b0 = buf * L

                def acc_body(t, accs):
                    out = []
                    for c in range(4):
                        a = accs[c]
                        a = a + rows_v[b0 + t, pl.ds(c * 16, 16)]
                        a = a + rows_v[b0 + HALF + t, pl.ds(c * 16, 16)]
                        out.append(a)
                    return tuple(out)

                zeros = tuple(jnp.zeros((16,), jnp.float32) for _ in range(4))
                accs = lax.fori_loop(0, HALF, acc_body, zeros, unroll=2)
                lr = g * GROUP + r
                for c in range(4):
                    out_v[lr, pl.ds(c * 16, 16)] = accs[c]
                return 0

            lax.fori_loop(0, GROUP, row_body, 0)

        pltpu.sync_copy(out_v, out_hbm.at[pl.ds(base, BPW)])

    return _pool(x2, emb_table)


CW = 16384
NCONV = (VOCAB + CW - 1) // CW
VOCAB2 = NCONV * CW


def _conv_body(t_ref, o_ref):
    t = t_ref[...].T
    o_ref[...] = jnp.concatenate([t[: CW // 2], t[CW // 2:]], axis=1)


def _convert_table(tT):
    out = pl.pallas_call(
        _conv_body,
        grid=(NCONV,),
        in_specs=[pl.BlockSpec((EI, CW), lambda s: (0, s))],
        out_specs=pl.BlockSpec((CW // 2, 128), lambda s: (s, 0)),
        out_shape=jax.ShapeDtypeStruct((VOCAB2 // 2, 128), jnp.float32),
    )(tT)
    return out.reshape(VOCAB2, EI)


def _remap_idx(i):
    h = CW // 2
    return (i // CW) * CW + 2 * (i % h) + (i // h) % 2


def _fc_body(p_ref, w_ref, b_ref, o_ref):
    e = p_ref[...] * (1.0 / L)
    y = jnp.dot(e, w_ref[...], preferred_element_type=jnp.float32) + b_ref[...]
    n = jnp.sqrt(jnp.sum(y * y, axis=-1, keepdims=True))
    o_ref[...] = y / jnp.maximum(n, 1e-12)


def _fc_norm(pooled, fc_W, fc_b2):
    TB = 1024
    return pl.pallas_call(
        _fc_body,
        grid=(B // TB,),
        in_specs=[
            pl.BlockSpec((TB, EI), lambda i: (i, 0)),
            pl.BlockSpec((EI, ED), lambda i: (0, 0)),
            pl.BlockSpec((1, ED), lambda i: (0, 0)),
        ],
        out_specs=pl.BlockSpec((TB, ED), lambda i: (i, 0)),
        out_shape=jax.ShapeDtypeStruct((B, ED), jnp.float32),
    )(pooled, fc_W, fc_b2)


def kernel(x, emb_table, fc_W, fc_b):
    xp = jnp.pad(_remap_idx(x.astype(jnp.int32)), ((0, 0), (0, LP - L)))
    pooled = _pooled_sc(xp, _convert_table(emb_table.T))
    return _fc_norm(pooled, fc_W, fc_b.reshape(1, ED))


# speedup vs baseline: 1.1913x; 1.0243x over previous
import functools

import jax
import jax.numpy as jnp
from jax import lax
from jax.experimental import pallas as pl
from jax.experimental.pallas import tpu as pltpu
from jax.experimental.pallas import tpu_sc as plsc

VOCAB = 1000000
EI = 64
ED = 128
B = 16384
L = 200
HALF = L // 2
LP = 256
NC, NS = 2, 16
NW = NC * NS
BPW = B // NW
GROUP = 64
NGROUPS = BPW // GROUP
NBUF = 4


def _pooled_sc(x2, emb_table):
    mesh = plsc.VectorSubcoreMesh(core_axis_name="c", subcore_axis_name="s")

    @functools.partial(
        pl.kernel,
        out_type=jax.ShapeDtypeStruct((B, EI), jnp.float32),
        mesh=mesh,
        scratch_types=[
            pltpu.VMEM((GROUP, LP), jnp.int32),
            pltpu.VMEM((NBUF * L, EI), jnp.float32),
            pltpu.VMEM((BPW, EI), jnp.float32),
            pltpu.SemaphoreType.DMA((NBUF,)),
        ],
        compiler_params=pltpu.CompilerParams(use_tc_tiling_on_sc=False),
    )
    def _pool(x_hbm, table_hbm, out_hbm, idx_v, rows_v, out_v, sems):
        wid = lax.axis_index("s") * NC + lax.axis_index("c")
        base = wid * BPW

        def fire(r):
            buf = lax.rem(r, NBUF)
            return pltpu.async_copy(
                table_hbm.at[idx_v.at[r, pl.ds(0, L)]],
                rows_v.at[pl.ds(buf * L, L)], sems.at[buf])

        for g in range(NGROUPS):
            pltpu.sync_copy(
                x_hbm.at[pl.ds(base + g * GROUP, GROUP)], idx_v)
            for r0 in range(NBUF - 1):
                fire(r0)

            def row_body(r, _):
                buf = lax.rem(r, NBUF)
                pltpu.make_async_copy(
                    table_hbm.at[idx_v.at[r, pl.ds(0, L)]],
                    rows_v.at[pl.ds(buf * L, L)], sems.at[buf]).wait()

                @pl.when(r + (NBUF - 1) < GROUP)
                def _():
                    fire(r + (NBUF - 1))

                b0 = buf * L

                def acc_body(t, accs):
                    out = []
                    for c in range(4):
                        a = accs[c]
                        a = a + rows_v[b0 + t, pl.ds(c * 16, 16)]
                        a = a + rows_v[b0 + HALF + t, pl.ds(c * 16, 16)]
                        out.append(a)
                    return tuple(out)

                zeros = tuple(jnp.zeros((16,), jnp.float32) for _ in range(4))
                accs = lax.fori_loop(0, HALF, acc_body, zeros, unroll=2)
                lr = g * GROUP + r
                for c in range(4):
                    out_v[lr, pl.ds(c * 16, 16)] = accs[c]
                return 0

            lax.fori_loop(0, GROUP, row_body, 0)

        pltpu.sync_copy(out_v, out_hbm.at[pl.ds(base, BPW)])

    return _pool(x2, emb_table)


CW = 32768
NCONV = (VOCAB + CW - 1) // CW
VOCAB2 = NCONV * CW


def _conv_body(t_ref, o_ref):
    t = t_ref[...].T
    o_ref[...] = jnp.concatenate([t[: CW // 2], t[CW // 2:]], axis=1)


def _convert_table(tT):
    out = pl.pallas_call(
        _conv_body,
        grid=(NCONV,),
        in_specs=[pl.BlockSpec((EI, CW), lambda s: (0, s))],
        out_specs=pl.BlockSpec((CW // 2, 128), lambda s: (s, 0)),
        out_shape=jax.ShapeDtypeStruct((VOCAB2 // 2, 128), jnp.float32),
    )(tT)
    return out.reshape(VOCAB2, EI)


def _remap_idx(i):
    h = CW // 2
    return (i // CW) * CW + 2 * (i % h) + (i // h) % 2


def _fc_body(p_ref, w_ref, b_ref, o_ref):
    e = p_ref[...] * (1.0 / L)
    y = jnp.dot(e, w_ref[...], preferred_element_type=jnp.float32) + b_ref[...]
    n = jnp.sqrt(jnp.sum(y * y, axis=-1, keepdims=True))
    o_ref[...] = y / jnp.maximum(n, 1e-12)


def _fc_norm(pooled, fc_W, fc_b2):
    TB = 1024
    return pl.pallas_call(
        _fc_body,
        grid=(B // TB,),
        in_specs=[
            pl.BlockSpec((TB, EI), lambda i: (i, 0)),
            pl.BlockSpec((EI, ED), lambda i: (0, 0)),
            pl.BlockSpec((1, ED), lambda i: (0, 0)),
        ],
        out_specs=pl.BlockSpec((TB, ED), lambda i: (i, 0)),
        out_shape=jax.ShapeDtypeStruct((B, ED), jnp.float32),
    )(pooled, fc_W, fc_b2)


def kernel(x, emb_table, fc_W, fc_b):
    xp = jnp.pad(_remap_idx(x.astype(jnp.int32)), ((0, 0), (0, LP - L)))
    pooled = _pooled_sc(xp, _convert_table(emb_table.T))
    return _fc_norm(pooled, fc_W, fc_b.reshape(1, ED))


# speedup vs baseline: 1.2159x; 1.0206x over previous
import functools

import jax
import jax.numpy as jnp
from jax import lax
from jax.experimental import pallas as pl
from jax.experimental.pallas import tpu as pltpu
from jax.experimental.pallas import tpu_sc as plsc

VOCAB = 1000000
EI = 64
ED = 128
B = 16384
L = 200
HALF = L // 2
LP = 256
NC, NS = 2, 16
NW = NC * NS
BPW = B // NW
GROUP = 128
NGROUPS = BPW // GROUP
NBUF = 4


def _pooled_sc(xp, table):
    mesh = plsc.VectorSubcoreMesh(core_axis_name="c", subcore_axis_name="s")

    @functools.partial(
        pl.kernel,
        out_type=jax.ShapeDtypeStruct((B, EI), jnp.float32),
        mesh=mesh,
        scratch_types=[
            pltpu.VMEM((GROUP, LP), jnp.int32),
            pltpu.VMEM((NBUF * L, EI), jnp.float32),
            pltpu.VMEM((BPW, EI), jnp.float32),
            pltpu.SemaphoreType.DMA((NBUF,)),
        ],
        compiler_params=pltpu.CompilerParams(use_tc_tiling_on_sc=False),
    )
    def _pool(x_hbm, table_hbm, out_hbm, idx_v, rows_v, out_v, sems):
        wid = lax.axis_index("s") * NC + lax.axis_index("c")
        base = wid * BPW

        def fire(r):
            buf = lax.rem(r, NBUF)
            return pltpu.async_copy(
                table_hbm.at[idx_v.at[r, pl.ds(0, L)]],
                rows_v.at[pl.ds(buf * L, L)], sems.at[buf])

        for g in range(NGROUPS):
            pltpu.sync_copy(
                x_hbm.at[pl.ds(base + g * GROUP, GROUP)], idx_v)
            for r0 in range(NBUF - 1):
                fire(r0)

            def row_body(r, _):
                buf = lax.rem(r, NBUF)
                pltpu.make_async_copy(
                    table_hbm.at[idx_v.at[r, pl.ds(0, L)]],
                    rows_v.at[pl.ds(buf * L, L)], sems.at[buf]).wait()

                @pl.when(r + (NBUF - 1) < GROUP)
                def _():
                    fire(r + (NBUF - 1))

                b0 = buf * L

                def acc_body(t, accs):
                    out = []
                    for c in range(4):
                        a = accs[c]
                        a = a + rows_v[b0 + t, pl.ds(c * 16, 16)]
                        a = a + rows_v[b0 + HALF + t, pl.ds(c * 16, 16)]
                        out.append(a)
                    return tuple(out)

                zeros = tuple(jnp.zeros((16,), jnp.float32) for _ in range(4))
                accs = lax.fori_loop(0, HALF, acc_body, zeros, unroll=2)
                lr = g * GROUP + r
                for c in range(4):
                    out_v[lr, pl.ds(c * 16, 16)] = accs[c]
                return 0

            lax.fori_loop(0, GROUP, row_body, 0)

        pltpu.sync_copy(out_v, out_hbm.at[pl.ds(base, BPW)])

    return _pool(xp, table)


CW = 32768
NCONV = (VOCAB + CW - 1) // CW
VOCAB2 = NCONV * CW


def _conv_body(t_ref, o_ref):
    t = t_ref[...].T
    o_ref[...] = jnp.concatenate([t[: CW // 2], t[CW // 2:]], axis=1)


def _convert_table(tT):
    out = pl.pallas_call(
        _conv_body,
        grid=(NCONV,),
        in_specs=[pl.BlockSpec((EI, CW), lambda s: (0, s))],
        out_specs=pl.BlockSpec((CW // 2, 128), lambda s: (s, 0)),
        out_shape=jax.ShapeDtypeStruct((VOCAB2 // 2, 128), jnp.float32),
    )(tT)
    return out.reshape(VOCAB2, EI)


def _remap_idx(i):
    h = CW // 2
    return (i // CW) * CW + 2 * (i % h) + (i // h) % 2


def _fc_body(p_ref, w_ref, b_ref, o_ref):
    e = p_ref[...] * (1.0 / L)
    y = jnp.dot(e, w_ref[...], preferred_element_type=jnp.float32) + b_ref[...]
    n = jnp.sqrt(jnp.sum(y * y, axis=-1, keepdims=True))
    o_ref[...] = y / jnp.maximum(n, 1e-12)


def _fc_norm(pooled, fc_W, fc_b2):
    TB = 1024
    return pl.pallas_call(
        _fc_body,
        grid=(B // TB,),
        in_specs=[
            pl.BlockSpec((TB, EI), lambda i: (i, 0)),
            pl.BlockSpec((EI, ED), lambda i: (0, 0)),
            pl.BlockSpec((1, ED), lambda i: (0, 0)),
        ],
        out_specs=pl.BlockSpec((TB, ED), lambda i: (i, 0)),
        out_shape=jax.ShapeDtypeStruct((B, ED), jnp.float32),
    )(pooled, fc_W, fc_b2)


def kernel(x, emb_table, fc_W, fc_b):
    xp = jnp.pad(_remap_idx(x.astype(jnp.int32)), ((0, 0), (0, LP - L)))
    pooled = _pooled_sc(xp, _convert_table(emb_table.T))
    return _fc_norm(pooled, fc_W, fc_b.reshape(1, ED))
